# baseline (speedup 1.0000x reference)
import functools

import jax
import jax.numpy as jnp
from jax import lax
from jax.experimental import pallas as pl
from jax.experimental.pallas import tpu as pltpu
from jax.experimental.pallas import tpu_sc as plsc

NC, NS, L = 2, 16, 16
NW = NC * NS
B = 16384
D = 32
V = 1000000
SCN = 8192
TCN = B - SCN
BPW = SCN // NW
CE = 4
NG = BPW // L
CPG = L // CE
NBUF = 3
EPG = 32


def _sc_body(uidx_hbm, vidx_hbm, ut_hbm, vt_hbm, w_hbm, b_hbm, out_hbm,
             uidx_v, vidx_v, ubuf_v, mbuf_v, wv_v, bv_v,
             out_v, sems):
    wid = lax.axis_index("s") * NC + lax.axis_index("c")
    base = wid * BPW

    pltpu.sync_copy(uidx_hbm.at[pl.ds(base, BPW)], uidx_v.at[pl.ds(0, BPW)])
    pltpu.sync_copy(vidx_hbm.at[pl.ds(base, BPW)], vidx_v.at[pl.ds(0, BPW)])
    pltpu.sync_copy(w_hbm, wv_v)
    pltpu.sync_copy(b_hbm, bv_v)

    def fire(c, par):
        uvec = uidx_v[pl.ds(c * CE, L)] & -128
        vvec = vidx_v[pl.ds(c * CE, L)] & -128
        for e in range(CE):
            uj = pl.multiple_of(uvec[e], 128)
            vj = pl.multiple_of(vvec[e], 128)
            pltpu.async_copy(ut_hbm.at[:, pl.ds(uj, 128)],
                             ubuf_v.at[par, e], sems.at[par])
            pltpu.async_copy(vt_hbm.at[:, pl.ds(vj, 128)],
                             mbuf_v.at[par, e], sems.at[par])

    def drain(par):
        for e in range(CE):
            pltpu.make_async_copy(ut_hbm.at[:, pl.ds(0, 128)],
                                  ubuf_v.at[par, e], sems.at[par]).wait()
            pltpu.make_async_copy(vt_hbm.at[:, pl.ds(0, 128)],
                                  mbuf_v.at[par, e], sems.at[par]).wait()

    iota = lax.iota(jnp.int32, L)
    d_lo = lax.iota(jnp.int32, L)
    d_hi = d_lo + L
    wv = wv_v[...]
    bv = bv_v[...]

    fire(0, 0)
    fire(1, 1)

    @pl.loop(0, NG)
    def _group(g):
        acc = jnp.zeros((L,), jnp.float32)
        for q in range(CPG):
            c = g * CPG + q
            par = lax.rem(c, NBUF)

            @pl.when(c + 2 < NG * CPG)
            def _():
                fire(c + 2, lax.rem(c + 2, NBUF))

            drain(par)
            ulanes = uidx_v[pl.ds(c * CE, L)] & 127
            vlanes = vidx_v[pl.ds(c * CE, L)] & 127
            a = acc
            for e in range(CE):
                ku = ulanes[e] + jnp.zeros((L,), jnp.int32)
                kv = vlanes[e] + jnp.zeros((L,), jnp.int32)
                u0 = plsc.load_gather(ubuf_v.at[par, e], [d_lo, ku])
                u1 = plsc.load_gather(ubuf_v.at[par, e], [d_hi, ku])
                m0 = plsc.load_gather(mbuf_v.at[par, e], [d_lo, kv])
                m1 = plsc.load_gather(mbuf_v.at[par, e], [d_hi, kv])
                s = jnp.sum(u0 * m0 + u1 * m1)
                lane = q * CE + e
                a = a + jnp.where(iota == lane, s, 0.0)
            acc = a
        z = acc * wv + bv
        sig = 1.0 / (1.0 + jnp.exp(-z))
        plsc.store_scatter(out_v, [g * L + iota], sig)

    pltpu.sync_copy(out_v, out_hbm.at[pl.ds(base, BPW)])


def _sc_half(uidx, vidx, utt, vtt, wv, bv):
    mesh = plsc.VectorSubcoreMesh(core_axis_name="c", subcore_axis_name="s")
    f = pl.kernel(
        _sc_body,
        out_type=jax.ShapeDtypeStruct((SCN,), jnp.float32),
        mesh=mesh,
        scratch_types=[
            pltpu.VMEM((BPW + L,), jnp.int32),
            pltpu.VMEM((BPW + L,), jnp.int32),
            pltpu.VMEM((NBUF, CE, D, 128), jnp.float32),
            pltpu.VMEM((NBUF, CE, D, 128), jnp.float32),
            pltpu.VMEM((L,), jnp.float32),
            pltpu.VMEM((L,), jnp.float32),
            pltpu.VMEM((BPW,), jnp.float32),
            pltpu.SemaphoreType.DMA((NBUF,)),
        ],
        compiler_params=pltpu.CompilerParams(
            needs_layout_passes=False, use_tc_tiling_on_sc=True),
    )
    return f(uidx, vidx, utt, vtt, wv, bv)


def _tc_body(ju_ref, ku_ref, jv_ref, kv_ref, wb_ref, *refs):
    ublks = refs[:EPG]
    mblks = refs[EPG:2 * EPG]
    out_ref = refs[2 * EPG]
    i = pl.program_id(0)
    lanes = lax.broadcasted_iota(jnp.int32, (D, 128), 1)
    dots = []
    for e in range(EPG):
        ku = ku_ref[i * EPG + e]
        kv = kv_ref[i * EPG + e]
        usel = jnp.where(lanes == ku, ublks[e][...], 0.0)
        msel = jnp.where(lanes == kv, mblks[e][...], 0.0)
        dots.append(jnp.sum(jnp.sum(usel, axis=1) * jnp.sum(msel, axis=1)))
    z = jnp.stack(dots) * wb_ref[0] + wb_ref[1]
    sig = 1.0 / (1.0 + jnp.exp(-z))
    out_ref[...] = jnp.broadcast_to(sig[:, None], (EPG, 128))


def _tc_half(uidx, vidx, utt, vtt, wb):
    ju = jnp.right_shift(uidx, 7)
    ku = uidx & 127
    jv = jnp.right_shift(vidx, 7)
    kv = vidx & 127

    def u_im(e):
        return lambda i, ju_r, ku_r, jv_r, kv_r, wb_r: (0, ju_r[i * EPG + e])

    def v_im(e):
        return lambda i, ju_r, ku_r, jv_r, kv_r, wb_r: (0, jv_r[i * EPG + e])

    def o_im(i, ju_r, ku_r, jv_r, kv_r, wb_r):
        return (i, 0)

    grid_spec = pltpu.PrefetchScalarGridSpec(
        num_scalar_prefetch=5,
        grid=(TCN // EPG,),
        in_specs=(
            [pl.BlockSpec((D, 128), u_im(e)) for e in range(EPG)]
            + [pl.BlockSpec((D, 128), v_im(e)) for e in range(EPG)]
        ),
        out_specs=pl.BlockSpec((EPG, 128), o_im),
    )
    f = pl.pallas_call(
        _tc_body,
        grid_spec=grid_spec,
        out_shape=jax.ShapeDtypeStruct((TCN, 128), jnp.float32),
        compiler_params=pltpu.CompilerParams(
            dimension_semantics=("arbitrary",)),
    )
    tabs = [utt] * EPG + [vtt] * EPG
    out = f(ju, ku, jv, kv, wb, *tabs)
    return out[:, 0]


def kernel(x, user_table, video_table, fc_w, fc_b):
    uidx = x[0]
    vidx = x[1]
    utt = user_table.T
    vtt = video_table.T
    w = fc_w.reshape(1).astype(jnp.float32)
    b = fc_b.reshape(1).astype(jnp.float32)
    wv = jnp.broadcast_to(w, (L,))
    bv = jnp.broadcast_to(b, (L,))
    wb = jnp.concatenate([w, b])

    sc_out = _sc_half(uidx[:SCN], vidx[:SCN], utt, vtt, wv, bv)
    tc_out = _tc_half(uidx[SCN:], vidx[SCN:], utt, vtt, wb)
    return jnp.concatenate([sc_out, tc_out]).reshape(B, 1)


# speedup vs baseline: 2.7053x; 2.7053x over previous
import jax
import jax.numpy as jnp
from jax import lax
from jax.experimental import pallas as pl
from jax.experimental.pallas import tpu as pltpu
from jax.experimental.pallas import tpu_sc as plsc

NC, NS, L = 2, 16, 16
NW = NC * NS
B = 16384
D = 32
BPW = B // NW
CE = 4
NG = BPW // L
CPG = L // CE
NBUF = 3


def _sc_body(uidx_hbm, vidx_hbm, ut_hbm, vt_hbm, w_hbm, b_hbm, out_hbm,
             uidx_v, vidx_v, ubuf_v, mbuf_v, wv_v, bv_v,
             out_v, sems):
    wid = lax.axis_index("s") * NC + lax.axis_index("c")
    base = wid * BPW

    pltpu.sync_copy(uidx_hbm.at[pl.ds(base, BPW)], uidx_v.at[pl.ds(0, BPW)])
    pltpu.sync_copy(vidx_hbm.at[pl.ds(base, BPW)], vidx_v.at[pl.ds(0, BPW)])
    pltpu.sync_copy(w_hbm, wv_v)
    pltpu.sync_copy(b_hbm, bv_v)

    def fire(c, par):
        uvec = uidx_v[pl.ds(c * CE, L)] & -128
        vvec = vidx_v[pl.ds(c * CE, L)] & -128
        for e in range(CE):
            uj = pl.multiple_of(uvec[e], 128)
            vj = pl.multiple_of(vvec[e], 128)
            for sb in range(D // 8):
                pltpu.async_copy(
                    ut_hbm.at[pl.ds(sb * 8, 8), pl.ds(uj, 128)],
                    ubuf_v.at[par, e, pl.ds(sb * 8, 8)], sems.at[par])
                pltpu.async_copy(
                    vt_hbm.at[pl.ds(sb * 8, 8), pl.ds(vj, 128)],
                    mbuf_v.at[par, e, pl.ds(sb * 8, 8)], sems.at[par])

    def drain(par):
        for e in range(CE):
            pltpu.make_async_copy(ut_hbm.at[:, pl.ds(0, 128)],
                                  ubuf_v.at[par, e], sems.at[par]).wait()
            pltpu.make_async_copy(vt_hbm.at[:, pl.ds(0, 128)],
                                  mbuf_v.at[par, e], sems.at[par]).wait()

    iota = lax.iota(jnp.int32, L)
    d_lo = lax.iota(jnp.int32, L)
    d_hi = d_lo + L
    wv = wv_v[...]
    bv = bv_v[...]

    fire(0, 0)
    fire(1, 1)

    @pl.loop(0, NG)
    def _group(g):
        acc = jnp.zeros((L,), jnp.float32)
        for q in range(CPG):
            c = g * CPG + q
            par = lax.rem(c, NBUF)

            @pl.when(c + 2 < NG * CPG)
            def _():
                fire(c + 2, lax.rem(c + 2, NBUF))

            drain(par)
            ulanes = uidx_v[pl.ds(c * CE, L)] & 127
            vlanes = vidx_v[pl.ds(c * CE, L)] & 127
            a = acc
            for e in range(CE):
                ku = ulanes[e] + jnp.zeros((L,), jnp.int32)
                kv = vlanes[e] + jnp.zeros((L,), jnp.int32)
                u0 = plsc.load_gather(ubuf_v.at[par, e], [d_lo, ku])
                u1 = plsc.load_gather(ubuf_v.at[par, e], [d_hi, ku])
                m0 = plsc.load_gather(mbuf_v.at[par, e], [d_lo, kv])
                m1 = plsc.load_gather(mbuf_v.at[par, e], [d_hi, kv])
                s = jnp.sum(u0 * m0 + u1 * m1)
                lane = q * CE + e
                a = a + jnp.where(iota == lane, s, 0.0)
            acc = a
        z = acc * wv + bv
        sig = 1.0 / (1.0 + jnp.exp(-z))
        plsc.store_scatter(out_v, [g * L + iota], sig)

    pltpu.sync_copy(out_v, out_hbm.at[pl.ds(base, BPW)])


def kernel(x, user_table, video_table, fc_w, fc_b):
    uidx = x[0]
    vidx = x[1]
    utt = user_table.T
    vtt = video_table.T
    wv = jnp.broadcast_to(fc_w.reshape(1), (L,)).astype(jnp.float32)
    bv = jnp.broadcast_to(fc_b.reshape(1), (L,)).astype(jnp.float32)
    mesh = plsc.VectorSubcoreMesh(core_axis_name="c", subcore_axis_name="s")
    f = pl.kernel(
        _sc_body,
        out_type=jax.ShapeDtypeStruct((B,), jnp.float32),
        mesh=mesh,
        scratch_types=[
            pltpu.VMEM((BPW + L,), jnp.int32),
            pltpu.VMEM((BPW + L,), jnp.int32),
            pltpu.VMEM((NBUF, CE, D, 128), jnp.float32),
            pltpu.VMEM((NBUF, CE, D, 128), jnp.float32),
            pltpu.VMEM((L,), jnp.float32),
            pltpu.VMEM((L,), jnp.float32),
            pltpu.VMEM((BPW,), jnp.float32),
            pltpu.SemaphoreType.DMA((NBUF,)),
        ],
        compiler_params=pltpu.CompilerParams(
            needs_layout_passes=False, use_tc_tiling_on_sc=True),
    )
    out = f(uidx, vidx, utt, vtt, wv, bv)
    return out.reshape(B, 1)
